# baseline (speedup 1.0000x reference)
import functools

import jax
import jax.numpy as jnp
from jax import lax
from jax.experimental import pallas as pl
from jax.experimental.pallas import tpu as pltpu
import jax.experimental.pallas.tpu_sc as plsc

N = 10000
E = 320000
D = 128
W = 144
NC = 2
NS = 16
NW = NC * NS
EPW = E // NW
C = 48
NFULL = EPW // C
CT = EPW - NFULL * C
TPAIRS = NFULL // 2 - 1
RPT = 624


def _pre_body(x_ref, beta_ref, xn_ref, xnw_ref, ws_ref):
    xb = x_ref[...]
    b = beta_ref[0, 0]
    n2 = jnp.sum(xb * xb, axis=1, keepdims=True)
    nrm = jnp.sqrt(n2)
    xn = xb / jnp.maximum(nrm, 1e-12)
    s2 = jnp.sum(xn * xn, axis=1, keepdims=True)
    ws = jnp.exp(b * s2 - jnp.abs(b))
    xn_ref[...] = xn
    xnw_ref[...] = jnp.concatenate(
        [xn, nrm, jnp.zeros((xb.shape[0], W - D - 1), jnp.float32)], axis=1)
    ws_ref[...] = ws


def _post_body(nf_ref, x_ref, ws_ref, out_ref):
    nf = nf_ref[...]
    msg = nf[0, :, 0:D] + nf[1, :, 0:D]
    den = nf[0, :, D:D + 1] + nf[1, :, D:D + 1]
    ws = ws_ref[...]
    xb = x_ref[...]
    out_ref[...] = jnp.tanh((msg + ws * xb) / (den + ws + 1e-16))


def _sc_body(src_hbm, dst_hbm, xn_hbm, xnw_hbm, prm_hbm, out_hbm,
             si0, di0, si1, di1, sit, dit,
             a0, a1, b0, b1, at_v, bt_v, s_v, prm_v, num_sh,
             sa0, sb0, sa1, sb1, ss0, sd0, ss1, sd1):
    cid = lax.axis_index("c")
    sid = lax.axis_index("s")
    wid = cid * NS + sid

    zero16 = jnp.zeros((16,), jnp.float32)

    def zrow(r, carry):
        for k in range(W // 16):
            s_v[r, pl.ds(k * 16, 16)] = zero16
        return carry

    lax.fori_loop(0, C, zrow, 0)

    base_r = sid * RPT
    for j in range(RPT // C):
        pltpu.sync_copy(s_v, num_sh.at[pl.ds(base_r + j * C, C)])

    @pl.when(sid == NS - 1)
    def _zero_tail():
        pltpu.sync_copy(s_v.at[pl.ds(0, N - NS * RPT)],
                        num_sh.at[pl.ds(NS * RPT, N - NS * RPT)])

    pltpu.sync_copy(prm_hbm, prm_v)
    plsc.subcore_barrier()

    prm_vec = prm_v[...]
    beta_s = prm_vec[0]
    c_s = prm_vec[1]
    lanes = lax.iota(jnp.int32, 16)
    mask0f = jnp.where(lanes == 0, 1.0, 0.0).astype(jnp.float32)

    ebase = wid * EPW

    def fetch_idx(c, si, di, ssem, dsem):
        base = ebase + c * C
        pltpu.async_copy(src_hbm.at[pl.ds(base, C)], si, ssem)
        pltpu.async_copy(dst_hbm.at[pl.ds(base, C)], di, dsem)

    def wait_idx(c, si, di, ssem, dsem):
        base = ebase + c * C
        pltpu.make_async_copy(src_hbm.at[pl.ds(base, C)], si, ssem).wait()
        pltpu.make_async_copy(dst_hbm.at[pl.ds(base, C)], di, dsem).wait()

    def fetch_rows(si, di, a, b, asem, bsem):
        pltpu.async_copy(xnw_hbm.at[si], a, asem)
        pltpu.async_copy(xn_hbm.at[di], b, bsem)

    def wait_rows(si, di, a, b, asem, bsem):
        pltpu.make_async_copy(xnw_hbm.at[si], a, asem).wait()
        pltpu.make_async_copy(xn_hbm.at[di], b, bsem).wait()

    def compute(a_v, b_v, n_edges):
        @plsc.parallel_loop(0, n_edges, 1, unroll=4)
        def edge_body(e):
            al = [a_v[e, pl.ds(k * 16, 16)] for k in range(8)]
            bl = [b_v[e, pl.ds(k * 16, 16)] for k in range(8)]
            tail = a_v[e, pl.ds(D, 16)]
            p0 = al[0] * bl[0] + al[1] * bl[1]
            p1 = al[2] * bl[2] + al[3] * bl[3]
            p2 = al[4] * bl[4] + al[5] * bl[5]
            p3 = al[6] * bl[6] + al[7] * bl[7]
            dot = jnp.sum((p0 + p1) + (p2 + p3))
            dvec = jnp.broadcast_to(dot, (16,))
            wvec = jnp.exp(beta_s * dvec - c_s)
            wp = wvec * tail[0]
            for k in range(8):
                s_v[e, pl.ds(k * 16, 16)] = al[k] * wp
            s_v[e, pl.ds(D, 16)] = wvec * mask0f

    def scatter(di):
        pltpu.sync_copy(s_v, num_sh.at[di], add=True)

    fetch_idx(0, si0, di0, ss0, sd0)
    wait_idx(0, si0, di0, ss0, sd0)
    fetch_rows(si0, di0, a0, b0, sa0, sb0)
    fetch_idx(1, si1, di1, ss1, sd1)

    def pair_body(t, carry):
        c0 = 2 * t
        wait_rows(si0, di0, a0, b0, sa0, sb0)
        wait_idx(c0 + 1, si1, di1, ss1, sd1)
        fetch_rows(si1, di1, a1, b1, sa1, sb1)
        compute(a0, b0, C)
        scatter(di0)
        fetch_idx(c0 + 2, si0, di0, ss0, sd0)
        wait_rows(si1, di1, a1, b1, sa1, sb1)
        compute(a1, b1, C)
        scatter(di1)
        wait_idx(c0 + 2, si0, di0, ss0, sd0)
        fetch_rows(si0, di0, a0, b0, sa0, sb0)
        fetch_idx(c0 + 3, si1, di1, ss1, sd1)
        return carry

    lax.fori_loop(0, TPAIRS, pair_body, 0)

    wait_rows(si0, di0, a0, b0, sa0, sb0)
    wait_idx(NFULL - 1, si1, di1, ss1, sd1)
    fetch_rows(si1, di1, a1, b1, sa1, sb1)
    compute(a0, b0, C)
    scatter(di0)
    wait_rows(si1, di1, a1, b1, sa1, sb1)
    compute(a1, b1, C)
    scatter(di1)

    tbase = ebase + NFULL * C
    pltpu.sync_copy(src_hbm.at[pl.ds(tbase, CT)], sit)
    pltpu.sync_copy(dst_hbm.at[pl.ds(tbase, CT)], dit)
    pltpu.async_copy(xnw_hbm.at[sit], at_v, sa0)
    pltpu.async_copy(xn_hbm.at[dit], bt_v, sb0)
    pltpu.make_async_copy(xnw_hbm.at[sit], at_v, sa0).wait()
    pltpu.make_async_copy(xn_hbm.at[dit], bt_v, sb0).wait()
    compute(at_v, bt_v, CT)
    pltpu.sync_copy(s_v.at[pl.ds(0, CT)], num_sh.at[dit], add=True)

    plsc.subcore_barrier()

    out_base = cid * N + sid * RPT
    pltpu.sync_copy(num_sh.at[pl.ds(sid * RPT, RPT)],
                    out_hbm.at[pl.ds(out_base, RPT)])

    @pl.when(sid == NS - 1)
    def _drain_tail():
        pltpu.sync_copy(num_sh.at[pl.ds(NS * RPT, N - NS * RPT)],
                        out_hbm.at[pl.ds(cid * N + NS * RPT, N - NS * RPT)])


_sc_edge = functools.partial(
    pl.kernel,
    out_type=jax.ShapeDtypeStruct((NC * N, W), jnp.float32),
    mesh=plsc.VectorSubcoreMesh(core_axis_name="c", subcore_axis_name="s"),
    compiler_params=pltpu.CompilerParams(
        use_tc_tiling_on_sc=False, needs_layout_passes=False),
    scratch_types=(
        [pltpu.VMEM((C,), jnp.int32)] * 4
        + [pltpu.VMEM((CT,), jnp.int32)] * 2
        + [pltpu.VMEM((C, W), jnp.float32)] * 2
        + [pltpu.VMEM((C, D), jnp.float32)] * 2
        + [pltpu.VMEM((CT, W), jnp.float32),
           pltpu.VMEM((CT, D), jnp.float32),
           pltpu.VMEM((C, W), jnp.float32),
           pltpu.VMEM((16,), jnp.float32),
           pltpu.VMEM_SHARED((N, W), jnp.float32)]
        + [pltpu.SemaphoreType.DMA] * 8
    ),
)(_sc_body)


@jax.jit
def kernel(x, edge_index, beta):
    src = edge_index[0]
    dst = edge_index[1]

    BR = 1000
    xn, xnw, wself = pl.pallas_call(
        _pre_body,
        grid=(N // BR,),
        in_specs=[
            pl.BlockSpec((BR, D), lambda i: (i, 0)),
            pl.BlockSpec((1, 1), lambda i: (0, 0)),
        ],
        out_specs=[
            pl.BlockSpec((BR, D), lambda i: (i, 0)),
            pl.BlockSpec((BR, W), lambda i: (i, 0)),
            pl.BlockSpec((BR, 1), lambda i: (i, 0)),
        ],
        out_shape=[
            jax.ShapeDtypeStruct((N, D), jnp.float32),
            jax.ShapeDtypeStruct((N, W), jnp.float32),
            jax.ShapeDtypeStruct((N, 1), jnp.float32),
        ],
    )(x, beta.reshape(1, 1))

    params = jnp.concatenate(
        [beta, jnp.abs(beta), jnp.zeros((14,), jnp.float32)])

    numfull = _sc_edge(src, dst, xn, xnw, params)

    out = pl.pallas_call(
        _post_body,
        grid=(N // BR,),
        in_specs=[
            pl.BlockSpec((NC, BR, W), lambda i: (0, i, 0)),
            pl.BlockSpec((BR, D), lambda i: (i, 0)),
            pl.BlockSpec((BR, 1), lambda i: (i, 0)),
        ],
        out_specs=pl.BlockSpec((BR, D), lambda i: (i, 0)),
        out_shape=jax.ShapeDtypeStruct((N, D), jnp.float32),
    )(numfull.reshape(NC, N, W), x, wself)
    return out


# speedup vs baseline: 1.0737x; 1.0737x over previous
import functools

import jax
import jax.numpy as jnp
from jax import lax
from jax.experimental import pallas as pl
from jax.experimental.pallas import tpu as pltpu
import jax.experimental.pallas.tpu_sc as plsc

N = 10000
E = 320000
D = 128
W = 144
NC = 2
NS = 16
NW = NC * NS
EPW = E // NW
C = 48
NFULL = EPW // C
CT = EPW - NFULL * C
TPAIRS = NFULL // 2 - 1
RPT = 624


def _pre_body(x_ref, beta_ref, xn_ref, xnw_ref, ws_ref):
    xb = x_ref[...]
    b = beta_ref[0, 0]
    n2 = jnp.sum(xb * xb, axis=1, keepdims=True)
    nrm = jnp.sqrt(n2)
    xn = xb / jnp.maximum(nrm, 1e-12)
    s2 = jnp.sum(xn * xn, axis=1, keepdims=True)
    ws = jnp.exp(b * s2 - jnp.abs(b))
    xn_ref[...] = xn
    xnw_ref[...] = jnp.concatenate(
        [xn, nrm, jnp.zeros((xb.shape[0], W - D - 1), jnp.float32)], axis=1)
    ws_ref[...] = ws


def _post_body(nf_ref, x_ref, ws_ref, out_ref):
    nf = nf_ref[...]
    msg = nf[0, :, 0:D] + nf[1, :, 0:D]
    den = nf[0, :, D:D + 1] + nf[1, :, D:D + 1]
    ws = ws_ref[...]
    xb = x_ref[...]
    out_ref[...] = jnp.tanh((msg + ws * xb) / (den + ws + 1e-16))


def _sc_body(src_hbm, dst_hbm, xn_hbm, xnw_hbm, prm_hbm, out_hbm,
             si0, di0, si1, di1, dis0, dis1, sit, dit,
             a0, a1, b0, b1, s0, s1, prm_v, num_sh,
             sa0, sb0, sa1, sb1, ss0, sd0, ss1, sd1, ssc0, ssc1):
    cid = lax.axis_index("c")
    sid = lax.axis_index("s")
    wid = cid * NS + sid

    zero16 = jnp.zeros((16,), jnp.float32)

    def zero_buf(s_v):
        def zrow(r, carry):
            for k in range(W // 16):
                s_v[r, pl.ds(k * 16, 16)] = zero16
            return carry
        lax.fori_loop(0, C, zrow, 0)

    zero_buf(s0)
    zero_buf(s1)

    base_r = sid * RPT
    for j in range(RPT // C):
        pltpu.sync_copy(s0, num_sh.at[pl.ds(base_r + j * C, C)])

    @pl.when(sid == NS - 1)
    def _zero_tail():
        pltpu.sync_copy(s0.at[pl.ds(0, N - NS * RPT)],
                        num_sh.at[pl.ds(NS * RPT, N - NS * RPT)])

    pltpu.sync_copy(prm_hbm, prm_v)
    plsc.subcore_barrier()

    prm_vec = prm_v[...]
    beta_s = prm_vec[0]
    c_s = prm_vec[1]
    lanes = lax.iota(jnp.int32, 16)
    mask0f = jnp.where(lanes == 0, 1.0, 0.0).astype(jnp.float32)

    ebase = wid * EPW

    def fetch_idx(c, si, di, ssem, dsem):
        base = ebase + c * C
        pltpu.async_copy(src_hbm.at[pl.ds(base, C)], si, ssem)
        pltpu.async_copy(dst_hbm.at[pl.ds(base, C)], di, dsem)

    def wait_idx(c, si, di, ssem, dsem):
        base = ebase + c * C
        pltpu.make_async_copy(src_hbm.at[pl.ds(base, C)], si, ssem).wait()
        pltpu.make_async_copy(dst_hbm.at[pl.ds(base, C)], di, dsem).wait()

    def fetch_rows(si, di, a, b, asem, bsem):
        pltpu.async_copy(xnw_hbm.at[si], a, asem)
        pltpu.async_copy(xn_hbm.at[di], b, bsem)

    def wait_rows(si, di, a, b, asem, bsem):
        pltpu.make_async_copy(xnw_hbm.at[si], a, asem).wait()
        pltpu.make_async_copy(xn_hbm.at[di], b, bsem).wait()

    def compute(a_v, b_v, s_v, n_edges):
        @plsc.parallel_loop(0, n_edges, 1, unroll=2)
        def edge_body(e):
            al = [a_v[e, pl.ds(k * 16, 16)] for k in range(8)]
            bl = [b_v[e, pl.ds(k * 16, 16)] for k in range(8)]
            tail = a_v[e, pl.ds(D, 16)]
            p0 = al[0] * bl[0] + al[1] * bl[1]
            p1 = al[2] * bl[2] + al[3] * bl[3]
            p2 = al[4] * bl[4] + al[5] * bl[5]
            p3 = al[6] * bl[6] + al[7] * bl[7]
            dot = jnp.sum((p0 + p1) + (p2 + p3))
            dvec = jnp.broadcast_to(dot, (16,))
            wvec = jnp.exp(beta_s * dvec - c_s)
            wp = wvec * tail[0]
            for k in range(8):
                s_v[e, pl.ds(k * 16, 16)] = al[k] * wp
            s_v[e, pl.ds(D, 16)] = wvec * mask0f

    def scatter(s_v, di, dis, sem):
        for g in range(C // 16):
            dis[pl.ds(g * 16, 16)] = di[pl.ds(g * 16, 16)]
        pltpu.async_copy(s_v, num_sh.at[dis], sem, add=True)

    def wait_scatter(s_v, dis, sem):
        pltpu.make_async_copy(s_v, num_sh.at[dis], sem).wait()

    fetch_idx(0, si0, di0, ss0, sd0)
    wait_idx(0, si0, di0, ss0, sd0)
    fetch_rows(si0, di0, a0, b0, sa0, sb0)
    fetch_idx(1, si1, di1, ss1, sd1)
    scatter(s0, di0, dis0, ssc0)
    scatter(s1, di0, dis1, ssc1)

    def pair_body(t, carry):
        c0 = 2 * t
        wait_rows(si0, di0, a0, b0, sa0, sb0)
        wait_idx(c0 + 1, si1, di1, ss1, sd1)
        fetch_rows(si1, di1, a1, b1, sa1, sb1)
        wait_scatter(s0, dis0, ssc0)
        compute(a0, b0, s0, C)
        scatter(s0, di0, dis0, ssc0)
        fetch_idx(c0 + 2, si0, di0, ss0, sd0)
        wait_rows(si1, di1, a1, b1, sa1, sb1)
        wait_scatter(s1, dis1, ssc1)
        compute(a1, b1, s1, C)
        scatter(s1, di1, dis1, ssc1)
        wait_idx(c0 + 2, si0, di0, ss0, sd0)
        fetch_rows(si0, di0, a0, b0, sa0, sb0)
        fetch_idx(c0 + 3, si1, di1, ss1, sd1)
        return carry

    lax.fori_loop(0, TPAIRS, pair_body, 0)

    wait_rows(si0, di0, a0, b0, sa0, sb0)
    wait_idx(NFULL - 1, si1, di1, ss1, sd1)
    fetch_rows(si1, di1, a1, b1, sa1, sb1)
    wait_scatter(s0, dis0, ssc0)
    compute(a0, b0, s0, C)
    scatter(s0, di0, dis0, ssc0)
    wait_rows(si1, di1, a1, b1, sa1, sb1)
    wait_scatter(s1, dis1, ssc1)
    compute(a1, b1, s1, C)
    scatter(s1, di1, dis1, ssc1)

    tbase = ebase + NFULL * C
    pltpu.sync_copy(src_hbm.at[pl.ds(tbase, CT)], sit)
    pltpu.sync_copy(dst_hbm.at[pl.ds(tbase, CT)], dit)
    pltpu.async_copy(xnw_hbm.at[sit], a1.at[pl.ds(0, CT)], sa0)
    pltpu.async_copy(xn_hbm.at[dit], b1.at[pl.ds(0, CT)], sb0)
    pltpu.make_async_copy(xnw_hbm.at[sit], a1.at[pl.ds(0, CT)], sa0).wait()
    pltpu.make_async_copy(xn_hbm.at[dit], b1.at[pl.ds(0, CT)], sb0).wait()
    wait_scatter(s0, dis0, ssc0)
    compute(a1, b1, s0, CT)
    pltpu.sync_copy(s0.at[pl.ds(0, CT)], num_sh.at[dit], add=True)
    wait_scatter(s1, dis1, ssc1)

    plsc.subcore_barrier()

    out_base = cid * N + sid * RPT
    pltpu.sync_copy(num_sh.at[pl.ds(sid * RPT, RPT)],
                    out_hbm.at[pl.ds(out_base, RPT)])

    @pl.when(sid == NS - 1)
    def _drain_tail():
        pltpu.sync_copy(num_sh.at[pl.ds(NS * RPT, N - NS * RPT)],
                        out_hbm.at[pl.ds(cid * N + NS * RPT, N - NS * RPT)])


_sc_edge = functools.partial(
    pl.kernel,
    out_type=jax.ShapeDtypeStruct((NC * N, W), jnp.float32),
    mesh=plsc.VectorSubcoreMesh(core_axis_name="c", subcore_axis_name="s"),
    compiler_params=pltpu.CompilerParams(
        use_tc_tiling_on_sc=False, needs_layout_passes=False),
    scratch_types=(
        [pltpu.VMEM((C,), jnp.int32)] * 6
        + [pltpu.VMEM((CT,), jnp.int32)] * 2
        + [pltpu.VMEM((C, W), jnp.float32)] * 2
        + [pltpu.VMEM((C, D), jnp.float32)] * 2
        + [pltpu.VMEM((C, W), jnp.float32)] * 2
        + [pltpu.VMEM((16,), jnp.float32),
           pltpu.VMEM_SHARED((N, W), jnp.float32)]
        + [pltpu.SemaphoreType.DMA] * 10
    ),
)(_sc_body)


@jax.jit
def kernel(x, edge_index, beta):
    src = edge_index[0]
    dst = edge_index[1]

    BR = 1000
    xn, xnw, wself = pl.pallas_call(
        _pre_body,
        grid=(N // BR,),
        in_specs=[
            pl.BlockSpec((BR, D), lambda i: (i, 0)),
            pl.BlockSpec((1, 1), lambda i: (0, 0)),
        ],
        out_specs=[
            pl.BlockSpec((BR, D), lambda i: (i, 0)),
            pl.BlockSpec((BR, W), lambda i: (i, 0)),
            pl.BlockSpec((BR, 1), lambda i: (i, 0)),
        ],
        out_shape=[
            jax.ShapeDtypeStruct((N, D), jnp.float32),
            jax.ShapeDtypeStruct((N, W), jnp.float32),
            jax.ShapeDtypeStruct((N, 1), jnp.float32),
        ],
    )(x, beta.reshape(1, 1))

    params = jnp.concatenate(
        [beta, jnp.abs(beta), jnp.zeros((14,), jnp.float32)])

    numfull = _sc_edge(src, dst, xn, xnw, params)

    out = pl.pallas_call(
        _post_body,
        grid=(N // BR,),
        in_specs=[
            pl.BlockSpec((NC, BR, W), lambda i: (0, i, 0)),
            pl.BlockSpec((BR, D), lambda i: (i, 0)),
            pl.BlockSpec((BR, 1), lambda i: (i, 0)),
        ],
        out_specs=pl.BlockSpec((BR, D), lambda i: (i, 0)),
        out_shape=jax.ShapeDtypeStruct((N, D), jnp.float32),
    )(numfull.reshape(NC, N, W), x, wself)
    return out


# speedup vs baseline: 1.1055x; 1.0296x over previous
import functools

import jax
import jax.numpy as jnp
from jax import lax
from jax.experimental import pallas as pl
from jax.experimental.pallas import tpu as pltpu
import jax.experimental.pallas.tpu_sc as plsc

N = 10000
E = 320000
D = 128
W = 144
NC = 2
NS = 16
NW = NC * NS
EPW = E // NW
C = 48
NFULL = EPW // C
CT = EPW - NFULL * C
TPAIRS = NFULL // 2 - 1
RPT = 624


def _pre_body(x_ref, beta_ref, xn_ref, xnw_ref, ws_ref):
    xb = x_ref[...]
    b = beta_ref[0, 0]
    n2 = jnp.sum(xb * xb, axis=1, keepdims=True)
    nrm = jnp.sqrt(n2)
    xn = xb / jnp.maximum(nrm, 1e-12)
    s2 = jnp.sum(xn * xn, axis=1, keepdims=True)
    ws = jnp.exp(b * s2 - jnp.abs(b))
    xn_ref[...] = xn
    xnw_ref[...] = jnp.concatenate(
        [xn, nrm, jnp.zeros((xb.shape[0], W - D - 1), jnp.float32)], axis=1)
    ws_ref[...] = ws


def _post_body(nf_ref, x_ref, ws_ref, out_ref):
    nf = nf_ref[...]
    msg = nf[0, :, 0:D] + nf[1, :, 0:D]
    den = nf[0, :, D:D + 1] + nf[1, :, D:D + 1]
    ws = ws_ref[...]
    xb = x_ref[...]
    out_ref[...] = jnp.tanh((msg + ws * xb) / (den + ws + 1e-16))


def _sc_body(src_hbm, dst_hbm, xn_hbm, xnw_hbm, prm_hbm, out_hbm,
             si0, di0, si1, di1, dis0, dis1, sit, dit,
             a0, a1, b0, b1, s0, s1, prm_v, num_sh,
             sa0, sb0, sa1, sb1, ss0, sd0, ss1, sd1, ssc0, ssc1):
    cid = lax.axis_index("c")
    sid = lax.axis_index("s")
    wid = cid * NS + sid

    zero16 = jnp.zeros((16,), jnp.float32)

    def zero_buf(s_v):
        def zrow(r, carry):
            for k in range(W // 16):
                s_v[r, pl.ds(k * 16, 16)] = zero16
            return carry
        lax.fori_loop(0, C, zrow, 0)

    zero_buf(s0)
    zero_buf(s1)

    base_r = sid * RPT
    for j in range(RPT // C):
        pltpu.sync_copy(s0, num_sh.at[pl.ds(base_r + j * C, C)])

    @pl.when(sid == NS - 1)
    def _zero_tail():
        pltpu.sync_copy(s0.at[pl.ds(0, N - NS * RPT)],
                        num_sh.at[pl.ds(NS * RPT, N - NS * RPT)])

    pltpu.sync_copy(prm_hbm, prm_v)
    plsc.subcore_barrier()

    prm_vec = prm_v[...]
    beta_s = prm_vec[0]
    c_s = prm_vec[1]
    lanes = lax.iota(jnp.int32, 16)
    mask0f = jnp.where(lanes == 0, 1.0, 0.0).astype(jnp.float32)

    ebase = wid * EPW

    def fetch_idx(c, si, di, ssem, dsem):
        base = ebase + c * C
        pltpu.async_copy(src_hbm.at[pl.ds(base, C)], si, ssem)
        pltpu.async_copy(dst_hbm.at[pl.ds(base, C)], di, dsem)

    def wait_idx(c, si, di, ssem, dsem):
        base = ebase + c * C
        pltpu.make_async_copy(src_hbm.at[pl.ds(base, C)], si, ssem).wait()
        pltpu.make_async_copy(dst_hbm.at[pl.ds(base, C)], di, dsem).wait()

    def fetch_rows(si, di, a, b, asem, bsem):
        pltpu.async_copy(xnw_hbm.at[si], a, asem)
        pltpu.async_copy(xn_hbm.at[di], b, bsem)

    def wait_rows(si, di, a, b, asem, bsem):
        pltpu.make_async_copy(xnw_hbm.at[si], a, asem).wait()
        pltpu.make_async_copy(xn_hbm.at[di], b, bsem).wait()

    def compute(a_v, b_v, s_v, n_edges):
        @plsc.parallel_loop(0, n_edges, 1, unroll=2)
        def edge_body(e):
            al = [a_v[e, pl.ds(k * 16, 16)] for k in range(8)]
            bl = []
            for m in range(4):
                bb = b_v[e, pl.ds(m * 32, 32)]
                lo, hi = plsc.unpack(bb, format=plsc.PackFormat.INTERLEAVED,
                                     preferred_element_type=jnp.float32)
                bl.append(lo)
                bl.append(hi)
            tail = a_v[e, pl.ds(D, 16)]
            p0 = al[0] * bl[0] + al[1] * bl[1]
            p1 = al[2] * bl[2] + al[3] * bl[3]
            p2 = al[4] * bl[4] + al[5] * bl[5]
            p3 = al[6] * bl[6] + al[7] * bl[7]
            dot = jnp.sum((p0 + p1) + (p2 + p3))
            dvec = jnp.broadcast_to(dot, (16,))
            wvec = jnp.exp(beta_s * dvec - c_s)
            wp = wvec * tail[0]
            for k in range(8):
                s_v[e, pl.ds(k * 16, 16)] = al[k] * wp
            s_v[e, pl.ds(D, 16)] = wvec * mask0f

    def scatter(s_v, di, dis, sem):
        for g in range(C // 16):
            dis[pl.ds(g * 16, 16)] = di[pl.ds(g * 16, 16)]
        pltpu.async_copy(s_v, num_sh.at[dis], sem, add=True)

    def wait_scatter(s_v, dis, sem):
        pltpu.make_async_copy(s_v, num_sh.at[dis], sem).wait()

    fetch_idx(0, si0, di0, ss0, sd0)
    wait_idx(0, si0, di0, ss0, sd0)
    fetch_rows(si0, di0, a0, b0, sa0, sb0)
    fetch_idx(1, si1, di1, ss1, sd1)
    scatter(s0, di0, dis0, ssc0)
    scatter(s1, di0, dis1, ssc1)

    def pair_body(t, carry):
        c0 = 2 * t
        wait_rows(si0, di0, a0, b0, sa0, sb0)
        wait_idx(c0 + 1, si1, di1, ss1, sd1)
        fetch_rows(si1, di1, a1, b1, sa1, sb1)
        wait_scatter(s0, dis0, ssc0)
        compute(a0, b0, s0, C)
        scatter(s0, di0, dis0, ssc0)
        fetch_idx(c0 + 2, si0, di0, ss0, sd0)
        wait_rows(si1, di1, a1, b1, sa1, sb1)
        wait_scatter(s1, dis1, ssc1)
        compute(a1, b1, s1, C)
        scatter(s1, di1, dis1, ssc1)
        wait_idx(c0 + 2, si0, di0, ss0, sd0)
        fetch_rows(si0, di0, a0, b0, sa0, sb0)
        fetch_idx(c0 + 3, si1, di1, ss1, sd1)
        return carry

    lax.fori_loop(0, TPAIRS, pair_body, 0)

    wait_rows(si0, di0, a0, b0, sa0, sb0)
    wait_idx(NFULL - 1, si1, di1, ss1, sd1)
    fetch_rows(si1, di1, a1, b1, sa1, sb1)
    wait_scatter(s0, dis0, ssc0)
    compute(a0, b0, s0, C)
    scatter(s0, di0, dis0, ssc0)
    wait_rows(si1, di1, a1, b1, sa1, sb1)
    wait_scatter(s1, dis1, ssc1)
    compute(a1, b1, s1, C)
    scatter(s1, di1, dis1, ssc1)

    tbase = ebase + NFULL * C
    pltpu.sync_copy(src_hbm.at[pl.ds(tbase, CT)], sit)
    pltpu.sync_copy(dst_hbm.at[pl.ds(tbase, CT)], dit)
    pltpu.async_copy(xnw_hbm.at[sit], a1.at[pl.ds(0, CT)], sa0)
    pltpu.async_copy(xn_hbm.at[dit], b1.at[pl.ds(0, CT)], sb0)
    pltpu.make_async_copy(xnw_hbm.at[sit], a1.at[pl.ds(0, CT)], sa0).wait()
    pltpu.make_async_copy(xn_hbm.at[dit], b1.at[pl.ds(0, CT)], sb0).wait()
    wait_scatter(s0, dis0, ssc0)
    compute(a1, b1, s0, CT)
    pltpu.sync_copy(s0.at[pl.ds(0, CT)], num_sh.at[dit], add=True)
    wait_scatter(s1, dis1, ssc1)

    plsc.subcore_barrier()

    out_base = cid * N + sid * RPT
    pltpu.sync_copy(num_sh.at[pl.ds(sid * RPT, RPT)],
                    out_hbm.at[pl.ds(out_base, RPT)])

    @pl.when(sid == NS - 1)
    def _drain_tail():
        pltpu.sync_copy(num_sh.at[pl.ds(NS * RPT, N - NS * RPT)],
                        out_hbm.at[pl.ds(cid * N + NS * RPT, N - NS * RPT)])


_sc_edge = functools.partial(
    pl.kernel,
    out_type=jax.ShapeDtypeStruct((NC * N, W), jnp.float32),
    mesh=plsc.VectorSubcoreMesh(core_axis_name="c", subcore_axis_name="s"),
    compiler_params=pltpu.CompilerParams(
        use_tc_tiling_on_sc=False, needs_layout_passes=False),
    scratch_types=(
        [pltpu.VMEM((C,), jnp.int32)] * 6
        + [pltpu.VMEM((CT,), jnp.int32)] * 2
        + [pltpu.VMEM((C, W), jnp.float32)] * 2
        + [pltpu.VMEM((C, D), jnp.bfloat16)] * 2
        + [pltpu.VMEM((C, W), jnp.float32)] * 2
        + [pltpu.VMEM((16,), jnp.float32),
           pltpu.VMEM_SHARED((N, W), jnp.float32)]
        + [pltpu.SemaphoreType.DMA] * 10
    ),
)(_sc_body)


@jax.jit
def kernel(x, edge_index, beta):
    src = edge_index[0]
    dst = edge_index[1]

    BR = 1000
    xn, xnw, wself = pl.pallas_call(
        _pre_body,
        grid=(N // BR,),
        in_specs=[
            pl.BlockSpec((BR, D), lambda i: (i, 0)),
            pl.BlockSpec((1, 1), lambda i: (0, 0)),
        ],
        out_specs=[
            pl.BlockSpec((BR, D), lambda i: (i, 0)),
            pl.BlockSpec((BR, W), lambda i: (i, 0)),
            pl.BlockSpec((BR, 1), lambda i: (i, 0)),
        ],
        out_shape=[
            jax.ShapeDtypeStruct((N, D), jnp.float32),
            jax.ShapeDtypeStruct((N, W), jnp.float32),
            jax.ShapeDtypeStruct((N, 1), jnp.float32),
        ],
    )(x, beta.reshape(1, 1))

    params = jnp.concatenate(
        [beta, jnp.abs(beta), jnp.zeros((14,), jnp.float32)])

    xnb = (xn.reshape(N, D // 32, 2, 16).transpose(0, 1, 3, 2)
           .reshape(N, D).astype(jnp.bfloat16))

    numfull = _sc_edge(src, dst, xnb, xnw, params)

    out = pl.pallas_call(
        _post_body,
        grid=(N // BR,),
        in_specs=[
            pl.BlockSpec((NC, BR, W), lambda i: (0, i, 0)),
            pl.BlockSpec((BR, D), lambda i: (i, 0)),
            pl.BlockSpec((BR, 1), lambda i: (i, 0)),
        ],
        out_specs=pl.BlockSpec((BR, D), lambda i: (i, 0)),
        out_shape=jax.ShapeDtypeStruct((N, D), jnp.float32),
    )(numfull.reshape(NC, N, W), x, wself)
    return out


# speedup vs baseline: 1.1106x; 1.0046x over previous
import functools

import jax
import jax.numpy as jnp
from jax import lax
from jax.experimental import pallas as pl
from jax.experimental.pallas import tpu as pltpu
import jax.experimental.pallas.tpu_sc as plsc

N = 10000
E = 320000
D = 128
W = 144
WA = 160
NC = 2
NS = 16
NW = NC * NS
EPW = E // NW
C = 48
NFULL = EPW // C
CT = EPW - NFULL * C
TPAIRS = NFULL // 2 - 1
RPT = 624


def _pre_body(x_ref, beta_ref, xn_ref, nrm_ref, ws_ref):
    xb = x_ref[...]
    b = beta_ref[0, 0]
    n2 = jnp.sum(xb * xb, axis=1, keepdims=True)
    nrm = jnp.sqrt(n2)
    xn = xb / jnp.maximum(nrm, 1e-12)
    s2 = jnp.sum(xn * xn, axis=1, keepdims=True)
    ws = jnp.exp(b * s2 - jnp.abs(b))
    xn_ref[...] = xn
    nrm_ref[...] = nrm
    ws_ref[...] = ws


def _post_body(nf_ref, x_ref, ws_ref, out_ref):
    nf = nf_ref[...]
    msg = nf[0, :, 0:D] + nf[1, :, 0:D]
    den = nf[0, :, D:D + 1] + nf[1, :, D:D + 1]
    ws = ws_ref[...]
    xb = x_ref[...]
    out_ref[...] = jnp.tanh((msg + ws * xb) / (den + ws + 1e-16))


def _sc_body(src_hbm, dst_hbm, xn_hbm, xnw_hbm, prm_hbm, out_hbm,
             si0, di0, si1, di1, dis0, dis1, sit, dit,
             a0, a1, b0, b1, s0, s1, prm_v, num_sh,
             sa0, sb0, sa1, sb1, ss0, sd0, ss1, sd1, ssc0, ssc1):
    cid = lax.axis_index("c")
    sid = lax.axis_index("s")
    wid = cid * NS + sid

    zero16 = jnp.zeros((16,), jnp.float32)

    def zero_buf(s_v):
        def zrow(r, carry):
            for k in range(W // 16):
                s_v[r, pl.ds(k * 16, 16)] = zero16
            return carry
        lax.fori_loop(0, C, zrow, 0)

    zero_buf(s0)
    zero_buf(s1)

    base_r = sid * RPT
    for j in range(RPT // C):
        pltpu.sync_copy(s0, num_sh.at[pl.ds(base_r + j * C, C)])

    @pl.when(sid == NS - 1)
    def _zero_tail():
        pltpu.sync_copy(s0.at[pl.ds(0, N - NS * RPT)],
                        num_sh.at[pl.ds(NS * RPT, N - NS * RPT)])

    pltpu.sync_copy(prm_hbm, prm_v)
    plsc.subcore_barrier()

    prm_vec = prm_v[...]
    beta_s = prm_vec[0]
    c_s = prm_vec[1]
    lanes = lax.iota(jnp.int32, 16)
    mask0f = jnp.where(lanes == 0, 1.0, 0.0).astype(jnp.float32)

    ebase = wid * EPW

    def fetch_idx(c, si, di, ssem, dsem):
        base = ebase + c * C
        pltpu.async_copy(src_hbm.at[pl.ds(base, C)], si, ssem)
        pltpu.async_copy(dst_hbm.at[pl.ds(base, C)], di, dsem)

    def wait_idx(c, si, di, ssem, dsem):
        base = ebase + c * C
        pltpu.make_async_copy(src_hbm.at[pl.ds(base, C)], si, ssem).wait()
        pltpu.make_async_copy(dst_hbm.at[pl.ds(base, C)], di, dsem).wait()

    def fetch_rows(si, di, a, b, asem, bsem):
        pltpu.async_copy(xnw_hbm.at[si], a, asem)
        pltpu.async_copy(xn_hbm.at[di], b, bsem)

    def wait_rows(si, di, a, b, asem, bsem):
        pltpu.make_async_copy(xnw_hbm.at[si], a, asem).wait()
        pltpu.make_async_copy(xn_hbm.at[di], b, bsem).wait()

    def compute(a_v, b_v, s_v, n_edges):
        @plsc.parallel_loop(0, n_edges, 1, unroll=2)
        def edge_body(e):
            al = []
            bl = []
            for m in range(4):
                aa = a_v[e, pl.ds(m * 32, 32)]
                lo, hi = plsc.unpack(aa, format=plsc.PackFormat.INTERLEAVED,
                                     preferred_element_type=jnp.float32)
                al.append(lo)
                al.append(hi)
                bb = b_v[e, pl.ds(m * 32, 32)]
                lo, hi = plsc.unpack(bb, format=plsc.PackFormat.INTERLEAVED,
                                     preferred_element_type=jnp.float32)
                bl.append(lo)
                bl.append(hi)
            tail, _ = plsc.unpack(a_v[e, pl.ds(D, 32)],
                                  format=plsc.PackFormat.INTERLEAVED,
                                  preferred_element_type=jnp.float32)
            p0 = al[0] * bl[0] + al[1] * bl[1]
            p1 = al[2] * bl[2] + al[3] * bl[3]
            p2 = al[4] * bl[4] + al[5] * bl[5]
            p3 = al[6] * bl[6] + al[7] * bl[7]
            dot = jnp.sum((p0 + p1) + (p2 + p3))
            dvec = jnp.broadcast_to(dot, (16,))
            wvec = jnp.exp(beta_s * dvec - c_s)
            wp = wvec * tail[0]
            for k in range(8):
                s_v[e, pl.ds(k * 16, 16)] = al[k] * wp
            s_v[e, pl.ds(D, 16)] = wvec * mask0f

    def scatter(s_v, di, dis, sem):
        for g in range(C // 16):
            dis[pl.ds(g * 16, 16)] = di[pl.ds(g * 16, 16)]
        pltpu.async_copy(s_v, num_sh.at[dis], sem, add=True)

    def wait_scatter(s_v, dis, sem):
        pltpu.make_async_copy(s_v, num_sh.at[dis], sem).wait()

    fetch_idx(0, si0, di0, ss0, sd0)
    wait_idx(0, si0, di0, ss0, sd0)
    fetch_rows(si0, di0, a0, b0, sa0, sb0)
    fetch_idx(1, si1, di1, ss1, sd1)
    scatter(s0, di0, dis0, ssc0)
    scatter(s1, di0, dis1, ssc1)

    def pair_body(t, carry):
        c0 = 2 * t
        wait_rows(si0, di0, a0, b0, sa0, sb0)
        wait_idx(c0 + 1, si1, di1, ss1, sd1)
        fetch_rows(si1, di1, a1, b1, sa1, sb1)
        wait_scatter(s0, dis0, ssc0)
        compute(a0, b0, s0, C)
        scatter(s0, di0, dis0, ssc0)
        fetch_idx(c0 + 2, si0, di0, ss0, sd0)
        wait_rows(si1, di1, a1, b1, sa1, sb1)
        wait_scatter(s1, dis1, ssc1)
        compute(a1, b1, s1, C)
        scatter(s1, di1, dis1, ssc1)
        wait_idx(c0 + 2, si0, di0, ss0, sd0)
        fetch_rows(si0, di0, a0, b0, sa0, sb0)
        fetch_idx(c0 + 3, si1, di1, ss1, sd1)
        return carry

    lax.fori_loop(0, TPAIRS, pair_body, 0)

    wait_rows(si0, di0, a0, b0, sa0, sb0)
    wait_idx(NFULL - 1, si1, di1, ss1, sd1)
    fetch_rows(si1, di1, a1, b1, sa1, sb1)
    wait_scatter(s0, dis0, ssc0)
    compute(a0, b0, s0, C)
    scatter(s0, di0, dis0, ssc0)
    wait_rows(si1, di1, a1, b1, sa1, sb1)
    wait_scatter(s1, dis1, ssc1)
    compute(a1, b1, s1, C)
    scatter(s1, di1, dis1, ssc1)

    tbase = ebase + NFULL * C
    pltpu.sync_copy(src_hbm.at[pl.ds(tbase, CT)], sit)
    pltpu.sync_copy(dst_hbm.at[pl.ds(tbase, CT)], dit)
    pltpu.async_copy(xnw_hbm.at[sit], a1.at[pl.ds(0, CT)], sa0)
    pltpu.async_copy(xn_hbm.at[dit], b1.at[pl.ds(0, CT)], sb0)
    pltpu.make_async_copy(xnw_hbm.at[sit], a1.at[pl.ds(0, CT)], sa0).wait()
    pltpu.make_async_copy(xn_hbm.at[dit], b1.at[pl.ds(0, CT)], sb0).wait()
    wait_scatter(s0, dis0, ssc0)
    compute(a1, b1, s0, CT)
    pltpu.sync_copy(s0.at[pl.ds(0, CT)], num_sh.at[dit], add=True)
    wait_scatter(s1, dis1, ssc1)

    plsc.subcore_barrier()

    out_base = cid * N + sid * RPT
    pltpu.sync_copy(num_sh.at[pl.ds(sid * RPT, RPT)],
                    out_hbm.at[pl.ds(out_base, RPT)])

    @pl.when(sid == NS - 1)
    def _drain_tail():
        pltpu.sync_copy(num_sh.at[pl.ds(NS * RPT, N - NS * RPT)],
                        out_hbm.at[pl.ds(cid * N + NS * RPT, N - NS * RPT)])


_sc_edge = functools.partial(
    pl.kernel,
    out_type=jax.ShapeDtypeStruct((NC * N, W), jnp.float32),
    mesh=plsc.VectorSubcoreMesh(core_axis_name="c", subcore_axis_name="s"),
    compiler_params=pltpu.CompilerParams(
        use_tc_tiling_on_sc=False, needs_layout_passes=False),
    scratch_types=(
        [pltpu.VMEM((C,), jnp.int32)] * 6
        + [pltpu.VMEM((CT,), jnp.int32)] * 2
        + [pltpu.VMEM((C, WA), jnp.bfloat16)] * 2
        + [pltpu.VMEM((C, D), jnp.bfloat16)] * 2
        + [pltpu.VMEM((C, W), jnp.float32)] * 2
        + [pltpu.VMEM((16,), jnp.float32),
           pltpu.VMEM_SHARED((N, W), jnp.float32)]
        + [pltpu.SemaphoreType.DMA] * 10
    ),
)(_sc_body)


@jax.jit
def kernel(x, edge_index, beta):
    src = edge_index[0]
    dst = edge_index[1]

    BR = 1000
    xn, nrm, wself = pl.pallas_call(
        _pre_body,
        grid=(N // BR,),
        in_specs=[
            pl.BlockSpec((BR, D), lambda i: (i, 0)),
            pl.BlockSpec((1, 1), lambda i: (0, 0)),
        ],
        out_specs=[
            pl.BlockSpec((BR, D), lambda i: (i, 0)),
            pl.BlockSpec((BR, 1), lambda i: (i, 0)),
            pl.BlockSpec((BR, 1), lambda i: (i, 0)),
        ],
        out_shape=[
            jax.ShapeDtypeStruct((N, D), jnp.float32),
            jax.ShapeDtypeStruct((N, 1), jnp.float32),
            jax.ShapeDtypeStruct((N, 1), jnp.float32),
        ],
    )(x, beta.reshape(1, 1))

    params = jnp.concatenate(
        [beta, jnp.abs(beta), jnp.zeros((14,), jnp.float32)])

    xnb = (xn.reshape(N, D // 32, 2, 16).transpose(0, 1, 3, 2)
           .reshape(N, D).astype(jnp.bfloat16))
    xnwb = jnp.concatenate(
        [xnb, nrm.astype(jnp.bfloat16),
         jnp.zeros((N, WA - D - 1), jnp.bfloat16)], axis=1)

    numfull = _sc_edge(src, dst, xnb, xnwb, params)

    out = pl.pallas_call(
        _post_body,
        grid=(N // BR,),
        in_specs=[
            pl.BlockSpec((NC, BR, W), lambda i: (0, i, 0)),
            pl.BlockSpec((BR, D), lambda i: (i, 0)),
            pl.BlockSpec((BR, 1), lambda i: (i, 0)),
        ],
        out_specs=pl.BlockSpec((BR, D), lambda i: (i, 0)),
        out_shape=jax.ShapeDtypeStruct((N, D), jnp.float32),
    )(numfull.reshape(NC, N, W), x, wself)
    return out
